# Initial kernel scaffold; baseline (speedup 1.0000x reference)
import jax
import jax.numpy as jnp
from jax.experimental import pallas as pl


def kernel(inputs, token_table, pos_table):
    raise NotImplementedError("write your pallas kernel here")


# speedup vs baseline: 3.3460x; 3.3460x over previous
import functools

import jax
import jax.numpy as jnp
from jax import lax
from jax.experimental import pallas as pl
from jax.experimental.pallas import tpu as pltpu
from jax.experimental.pallas import tpu_sc as plsc

D = 64
SEQ = 200
LANES = 16


@functools.partial(jax.jit, static_argnames=())
def kernel(inputs, token_table, pos_table):
    B, S = inputs.shape
    N = B * S
    idx = inputs.reshape(N).astype(jnp.int32)

    info = plsc.get_sparse_core_info()
    nw = info.num_cores * info.num_subcores
    per_w = N // nw
    n_chunks = per_w // SEQ

    mesh = plsc.VectorSubcoreMesh(core_axis_name="c", subcore_axis_name="s")

    @functools.partial(
        pl.kernel,
        mesh=mesh,
        out_type=jax.ShapeDtypeStruct((N, D), jnp.float32),
        compiler_params=pltpu.CompilerParams(use_tc_tiling_on_sc=False),
        scratch_types=[
            pltpu.VMEM((per_w,), jnp.int32),
            pltpu.VMEM((SEQ, D), jnp.float32),
            pltpu.VMEM((SEQ, D), jnp.float32),
            pltpu.SemaphoreType.DMA,
            pltpu.SemaphoreType.DMA,
        ],
    )
    def sc_embed(table_hbm, idx_hbm, pos_hbm, out_hbm,
                 idx_v, pos_v, rows_v, sem_a, sem_b):
        wid = lax.axis_index("s") * info.num_cores + lax.axis_index("c")
        base = wid * per_w
        pltpu.sync_copy(idx_hbm.at[pl.ds(base, per_w)], idx_v)
        pltpu.sync_copy(pos_hbm, pos_v)

        def chunk(g, carry):
            off = pl.multiple_of(g * SEQ, 8)
            cp_a = pltpu.async_copy(
                table_hbm.at[idx_v.at[pl.ds(off, 104)]],
                rows_v.at[pl.ds(0, 104)], sem_a)
            cp_b = pltpu.async_copy(
                table_hbm.at[idx_v.at[pl.ds(off + 104, 96)]],
                rows_v.at[pl.ds(104, 96)], sem_b)
            cp_a.wait()
            cp_b.wait()

            def row(r, c):
                for j in range(D // LANES):
                    sl = pl.ds(j * LANES, LANES)
                    rows_v[r, sl] = rows_v[r, sl] + pos_v[r, sl]
                return c

            lax.fori_loop(0, SEQ, row, 0)
            pltpu.sync_copy(rows_v, out_hbm.at[pl.ds(base + off, SEQ)])
            return carry

        lax.fori_loop(0, n_chunks, chunk, 0)

    out = sc_embed(token_table, idx, pos_table)
    return out.reshape(B, S, D)


# speedup vs baseline: 4.2423x; 1.2678x over previous
import functools

import jax
import jax.numpy as jnp
from jax import lax
from jax.experimental import pallas as pl
from jax.experimental.pallas import tpu as pltpu
from jax.experimental.pallas import tpu_sc as plsc

D = 64
SEQ = 200
LANES = 16
NBUF = 4


@jax.jit
def kernel(inputs, token_table, pos_table):
    B, S = inputs.shape
    N = B * S
    idx = inputs.reshape(N).astype(jnp.int32)

    info = plsc.get_sparse_core_info()
    nw = info.num_cores * info.num_subcores
    per_w = N // nw
    n_chunks = per_w // SEQ

    mesh = plsc.VectorSubcoreMesh(core_axis_name="c", subcore_axis_name="s")

    @functools.partial(
        pl.kernel,
        mesh=mesh,
        out_type=jax.ShapeDtypeStruct((N, D), jnp.float32),
        compiler_params=pltpu.CompilerParams(use_tc_tiling_on_sc=False),
        scratch_types=[
            pltpu.VMEM((per_w,), jnp.int32),
            pltpu.VMEM((SEQ, D), jnp.float32),
            pltpu.VMEM((NBUF, SEQ, D), jnp.float32),
        ] + [pltpu.SemaphoreType.DMA] * (2 * NBUF),
    )
    def sc_embed(table_hbm, idx_hbm, pos_hbm, out_hbm,
                 idx_v, pos_v, rows_v, *sems):
        gsem = sems[:NBUF]
        ssem = sems[NBUF:]
        wid = lax.axis_index("s") * info.num_cores + lax.axis_index("c")
        base = wid * per_w
        pltpu.sync_copy(idx_hbm.at[pl.ds(base, per_w)], idx_v)
        pltpu.sync_copy(pos_hbm, pos_v)

        def gather_copies(goff, slot, sem):
            buf = rows_v.at[slot]
            return (
                pltpu.make_async_copy(
                    table_hbm.at[idx_v.at[pl.ds(goff, 104)]],
                    buf.at[pl.ds(0, 104)], sem),
                pltpu.make_async_copy(
                    table_hbm.at[idx_v.at[pl.ds(goff + 104, 96)]],
                    buf.at[pl.ds(104, 96)], sem),
            )

        def store_copy(goff, slot, sem):
            return pltpu.make_async_copy(
                rows_v.at[slot], out_hbm.at[pl.ds(base + goff, SEQ)], sem)

        def start_gather(g, slot):
            goff = pl.multiple_of(g * SEQ, 8)
            for cp in gather_copies(goff, slot, gsem[slot]):
                cp.start()

        start_gather(0, 0)
        start_gather(1, 1)

        def quad(i, carry):
            for k in range(NBUF):
                g = i * NBUF + k
                goff = pl.multiple_of(g * SEQ, 8)
                buf = rows_v.at[k]
                for cp in gather_copies(goff, k, gsem[k]):
                    cp.wait()
                k2 = (k + 2) % NBUF

                @pl.when(g + 2 < n_chunks)
                def _():
                    @pl.when(g >= 2)
                    def _():
                        store_copy(0, k2, ssem[k2]).wait()
                    start_gather(g + 2, k2)

                def row(r, c):
                    for j in range(D // LANES):
                        sl = pl.ds(j * LANES, LANES)
                        plsc.addupdate(buf.at[r, sl], pos_v[r, sl])
                    return c

                lax.fori_loop(0, SEQ, row, 0, unroll=2)
                store_copy(goff, k, ssem[k]).start()
            return carry

        lax.fori_loop(0, n_chunks // NBUF, quad, 0)
        for k in range(NBUF):
            store_copy(0, k, ssem[k]).wait()

    out = sc_embed(token_table, idx, pos_table)
    return out.reshape(B, S, D)
